# Initial kernel scaffold; baseline (speedup 1.0000x reference)
import jax
import jax.numpy as jnp
from jax.experimental import pallas as pl


def kernel(orin_out, labels):
    raise NotImplementedError("write your pallas kernel here")


# speedup vs baseline: 1.0660x; 1.0660x over previous
import jax
import jax.numpy as jnp
from jax.experimental import pallas as pl

_MARGIN_S = 64.0
_MARGIN_M = 0.35
_BC = 2048


def _margin_block(labels_ref, x_ref, o_ref):
    j = pl.program_id(0)
    x = x_ref[...]
    labels = labels_ref[...]
    col0 = j * _BC
    cols = col0 + jax.lax.broadcasted_iota(jnp.int32, x.shape, 1)
    mask = cols == labels
    o_ref[...] = x * _MARGIN_S - jnp.where(mask, _MARGIN_S * _MARGIN_M, 0.0)


def kernel(orin_out, labels):
    b, n = orin_out.shape
    labels2d = labels.astype(jnp.int32).reshape(b, 1)
    grid = (pl.cdiv(n, _BC),)
    return pl.pallas_call(
        _margin_block,
        grid=grid,
        in_specs=[
            pl.BlockSpec((b, 1), lambda j: (0, 0)),
            pl.BlockSpec((b, _BC), lambda j: (0, j)),
        ],
        out_specs=pl.BlockSpec((b, _BC), lambda j: (0, j)),
        out_shape=jax.ShapeDtypeStruct((b, n), jnp.float32),
    )(labels2d, orin_out)
